# Initial kernel scaffold; baseline (speedup 1.0000x reference)
import jax
import jax.numpy as jnp
from jax import lax
from jax.experimental import pallas as pl
from jax.experimental.pallas import tpu as pltpu
from jax.experimental.pallas import tpu_sc as plsc

P, L, A = 32, 256, 28
T = 24
N_TABLES, BINS = 40, 36
NSTEP = L // 16

OFF_UP = 0
OFF_PRO = OFF_UP + T
OFF_RTAB = OFF_PRO + T
OFF_DOWN = OFF_RTAB + 2 * T
OFF_TOR = OFF_DOWN + 2 * T * A
META_LEN = OFF_TOR + 8 * T


def _sc_gather_body(coords_hbm, offs_hbm, bt_hbm, irc_hbm, meta_hbm, par_hbm,
                    pts_out, tib_out,
                    c_v, offs_v, bt_v, irc_v, meta_v, par_v, obuf_v, tib_v):
    cid = lax.axis_index("c")
    sid = lax.axis_index("s")
    wid = sid * 2 + cid
    pltpu.sync_copy(coords_hbm.at[wid], c_v)
    pltpu.sync_copy(offs_hbm.at[wid], offs_v)
    pltpu.sync_copy(bt_hbm.at[wid], bt_v)
    pltpu.sync_copy(irc_hbm.at[wid], irc_v)
    pltpu.sync_copy(meta_hbm, meta_v)
    pltpu.sync_copy(par_hbm, par_v)

    iota = lax.iota(jnp.int32, 16)
    for s in range(NSTEP):
        sl = pl.ds(s * 16, 16)
        bt = bt_v[sl]
        off = offs_v[sl]
        up = plsc.load_gather(meta_v, [bt + OFF_UP])
        ib = (iota + s * 16) * 4 + up * 2
        nb = plsc.load_gather(irc_v, [ib])
        nc = plsc.load_gather(irc_v, [ib + 1])
        nbt = plsc.load_gather(bt_v, [nb])
        noff = plsc.load_gather(offs_v, [nb])
        down = plsc.load_gather(meta_v, [OFF_DOWN + (nbt * 2 + nc) * A])
        inter = noff + down
        ipro = plsc.load_gather(meta_v, [OFF_PRO + nbt])
        ti = plsc.load_gather(meta_v, [OFF_RTAB + bt * 2 + ipro])
        tib_v[sl] = ti * (BINS * BINS)
        p4 = ti * 4
        for k in range(4):
            obuf_v[pl.ds((24 + k) * L + s * 16, 16)] = plsc.load_gather(par_v, [p4 + k])
        tor8 = bt * 8 + OFF_TOR
        for t in range(2):
            for j in range(4):
                ta = plsc.load_gather(meta_v, [tor8 + t * 4 + j])
                gi = jnp.where(ta >= 0, off + ta, inter)
                g3 = gi * 3
                row = (t * 4 + j) * 3
                obuf_v[pl.ds(row * L + s * 16, 16)] = plsc.load_gather(c_v, [g3])
                obuf_v[pl.ds((row + 1) * L + s * 16, 16)] = plsc.load_gather(c_v, [g3 + 1])
                obuf_v[pl.ds((row + 2) * L + s * 16, 16)] = plsc.load_gather(c_v, [g3 + 2])
    pltpu.sync_copy(obuf_v, pts_out.at[wid])
    pltpu.sync_copy(tib_v, tib_out.at[wid])


def _dot0(t):
    return (t[0] + t[1]) + t[2]


def _dihedral_rows(pc):
    p0, p1, p2, p3 = pc[0:3], pc[3:6], pc[6:9], pc[9:12]
    b0 = [p0[i] - p1[i] for i in range(3)]
    b1 = [p2[i] - p1[i] for i in range(3)]
    b2 = [p3[i] - p2[i] for i in range(3)]
    ss = _dot0([b1[i] * b1[i] for i in range(3)])
    den = jnp.sqrt(ss) + jnp.float32(1e-8)
    b1n = [b1[i] / den for i in range(3)]
    s0 = _dot0([b0[i] * b1n[i] for i in range(3)])
    v = [b0[i] - s0 * b1n[i] for i in range(3)]
    s2 = _dot0([b2[i] * b1n[i] for i in range(3)])
    w = [b2[i] - s2 * b1n[i] for i in range(3)]
    x = _dot0([v[i] * w[i] for i in range(3)])
    cr = [b1n[(i + 1) % 3] * v[(i + 2) % 3] - b1n[(i + 2) % 3] * v[(i + 1) % 3]
          for i in range(3)]
    y = _dot0([cr[i] * w[i] for i in range(3)])
    return jnp.arctan2(y, x)


def _tc_math_body(pts_ref, tib_ref, wts_ref, gidx_ref):
    angs = []
    for t in range(2):
        pc = [pts_ref[0, t * 12 + r, :] for r in range(12)]
        angs.append(_dihedral_rows(pc))
    phi, psi = angs
    prm = [pts_ref[0, 24 + k, :] for k in range(4)]
    fi = (phi - prm[0]) / prm[2]
    fj = (psi - prm[1]) / prm[3]
    i0f = jnp.floor(fi)
    j0f = jnp.floor(fj)
    a = fi - i0f
    b = fj - j0f
    i0 = jnp.mod(i0f.astype(jnp.int32), BINS)
    j0 = jnp.mod(j0f.astype(jnp.int32), BINS)
    i1 = jnp.mod(i0 + 1, BINS)
    j1 = jnp.mod(j0 + 1, BINS)
    tib = tib_ref[0, 0, :]
    wts_ref[0, 0, :] = (1 - a) * (1 - b)
    wts_ref[0, 1, :] = a * (1 - b)
    wts_ref[0, 2, :] = (1 - a) * b
    wts_ref[0, 3, :] = a * b
    gidx_ref[0, 0, :] = tib + i0 * BINS + j0
    gidx_ref[0, 1, :] = tib + i1 * BINS + j0
    gidx_ref[0, 2, :] = tib + i0 * BINS + j1
    gidx_ref[0, 3, :] = tib + i1 * BINS + j1


def _sc_combine_body(gidx_hbm, wts_hbm, rama_hbm, out_hbm,
                     gi_v, wt_v, rama_v, out_v):
    cid = lax.axis_index("c")
    sid = lax.axis_index("s")
    wid = sid * 2 + cid
    pltpu.sync_copy(gidx_hbm.at[wid], gi_v)
    pltpu.sync_copy(wts_hbm.at[wid], wt_v)
    pltpu.sync_copy(rama_hbm, rama_v)
    acc = jnp.zeros((16,), jnp.float32)
    for s in range(NSTEP):
        vals = []
        for k in range(4):
            g = gi_v[pl.ds(k * L + s * 16, 16)]
            wk = wt_v[pl.ds(k * L + s * 16, 16)]
            vals.append(plsc.load_gather(rama_v, [g]) * wk)
        acc = acc + ((vals[0] + vals[1]) + (vals[2] + vals[3]))
    tot = jnp.sum(acc)
    out_v[...] = jnp.full((16,), tot, jnp.float32)
    pltpu.sync_copy(out_v, out_hbm.at[wid])


def kernel(coords, pose_stack_block_coord_offset, pose_stack_block_type,
           pose_stack_inter_residue_connections, bt_atom_downstream_of_conn,
           bt_rama_table, bt_upper_conn_ind, bt_is_pro, bt_rama_torsion_atoms,
           rama_tables, table_params):
    coords2 = coords.reshape(P, L * A * 3)
    offs2 = pose_stack_block_coord_offset.astype(jnp.int32)
    bt2 = pose_stack_block_type.astype(jnp.int32)
    irc2 = pose_stack_inter_residue_connections.astype(jnp.int32).reshape(P, L * 4)
    meta = jnp.concatenate([
        bt_upper_conn_ind.astype(jnp.int32),
        bt_is_pro.astype(jnp.int32),
        bt_rama_table.astype(jnp.int32).reshape(-1),
        bt_atom_downstream_of_conn.astype(jnp.int32).reshape(-1),
        bt_rama_torsion_atoms.astype(jnp.int32).reshape(-1),
    ])
    parflat = table_params.astype(jnp.float32).reshape(-1)
    ramaflat = rama_tables.astype(jnp.float32).reshape(-1)

    mesh = plsc.VectorSubcoreMesh(core_axis_name="c", subcore_axis_name="s",
                                  num_cores=2, num_subcores=16)

    gather_stage = pl.kernel(
        _sc_gather_body,
        out_type=(jax.ShapeDtypeStruct((P, 28 * L), jnp.float32),
                  jax.ShapeDtypeStruct((P, L), jnp.int32)),
        mesh=mesh,
        scratch_types=[
            pltpu.VMEM((L * A * 3,), jnp.float32),
            pltpu.VMEM((L,), jnp.int32),
            pltpu.VMEM((L,), jnp.int32),
            pltpu.VMEM((L * 4,), jnp.int32),
            pltpu.VMEM((META_LEN,), jnp.int32),
            pltpu.VMEM((4 * N_TABLES,), jnp.float32),
            pltpu.VMEM((28 * L,), jnp.float32),
            pltpu.VMEM((L,), jnp.int32),
        ],
    )
    pts, tib = gather_stage(coords2, offs2, bt2, irc2, meta, parflat)

    wts, gidx = pl.pallas_call(
        _tc_math_body,
        grid=(P,),
        in_specs=[
            pl.BlockSpec((1, 28, L), lambda i: (i, 0, 0)),
            pl.BlockSpec((1, 1, L), lambda i: (i, 0, 0)),
        ],
        out_specs=[
            pl.BlockSpec((1, 4, L), lambda i: (i, 0, 0)),
            pl.BlockSpec((1, 4, L), lambda i: (i, 0, 0)),
        ],
        out_shape=[
            jax.ShapeDtypeStruct((P, 4, L), jnp.float32),
            jax.ShapeDtypeStruct((P, 4, L), jnp.int32),
        ],
    )(pts.reshape(P, 28, L), tib.reshape(P, 1, L))

    combine_stage = pl.kernel(
        _sc_combine_body,
        out_type=jax.ShapeDtypeStruct((P, 16), jnp.float32),
        mesh=mesh,
        scratch_types=[
            pltpu.VMEM((4 * L,), jnp.int32),
            pltpu.VMEM((4 * L,), jnp.float32),
            pltpu.VMEM((N_TABLES * BINS * BINS,), jnp.float32),
            pltpu.VMEM((16,), jnp.float32),
        ],
    )
    out = combine_stage(gidx.reshape(P, 4 * L), wts.reshape(P, 4 * L), ramaflat)
    return out[:, 0]


# speedup vs baseline: 6.7203x; 6.7203x over previous
import jax
import jax.numpy as jnp
from jax import lax
from jax.experimental import pallas as pl
from jax.experimental.pallas import tpu as pltpu
from jax.experimental.pallas import tpu_sc as plsc

P, L, A = 32, 256, 28
T = 24
N_TABLES, BINS = 40, 36
NSTEP = L // 16

OFF_UP = 0
OFF_PRO = OFF_UP + T
OFF_RTAB = OFF_PRO + T
OFF_DOWN = OFF_RTAB + 2 * T
OFF_TOR = OFF_DOWN + 2 * T * A
META_LEN = OFF_TOR + 8 * T


def _sc_gather_body(coords_hbm, offs_hbm, bt_hbm, irc_hbm, meta_hbm, par_hbm,
                    pts_out, tib_out,
                    c_v, offs_v, bt_v, irc_v, meta_v, par_v, obuf_v, tib_v):
    cid = lax.axis_index("c")
    sid = lax.axis_index("s")
    wid = sid * 2 + cid
    pltpu.sync_copy(coords_hbm.at[wid], c_v)
    pltpu.sync_copy(offs_hbm.at[wid], offs_v)
    pltpu.sync_copy(bt_hbm.at[wid], bt_v)
    pltpu.sync_copy(irc_hbm.at[wid], irc_v)
    pltpu.sync_copy(meta_hbm, meta_v)
    pltpu.sync_copy(par_hbm, par_v)

    iota = lax.iota(jnp.int32, 16)
    for s in range(NSTEP):
        sl = pl.ds(s * 16, 16)
        bt = bt_v[sl]
        off = offs_v[sl]
        up = plsc.load_gather(meta_v, [bt + OFF_UP])
        ib = (iota + s * 16) * 4 + up * 2
        nb = plsc.load_gather(irc_v, [ib])
        nc = plsc.load_gather(irc_v, [ib + 1])
        nbt = plsc.load_gather(bt_v, [nb])
        noff = plsc.load_gather(offs_v, [nb])
        down = plsc.load_gather(meta_v, [OFF_DOWN + (nbt * 2 + nc) * A])
        inter = noff + down
        ipro = plsc.load_gather(meta_v, [OFF_PRO + nbt])
        ti = plsc.load_gather(meta_v, [OFF_RTAB + bt * 2 + ipro])
        tib_v[sl] = ti * (BINS * BINS)
        p4 = ti * 4
        for k in range(4):
            obuf_v[pl.ds((24 + k) * L + s * 16, 16)] = plsc.load_gather(par_v, [p4 + k])
        tor8 = bt * 8 + OFF_TOR
        for t in range(2):
            for j in range(4):
                ta = plsc.load_gather(meta_v, [tor8 + t * 4 + j])
                gi = jnp.where(ta >= 0, off + ta, inter)
                g3 = gi * 3
                row = (t * 4 + j) * 3
                obuf_v[pl.ds(row * L + s * 16, 16)] = plsc.load_gather(c_v, [g3])
                obuf_v[pl.ds((row + 1) * L + s * 16, 16)] = plsc.load_gather(c_v, [g3 + 1])
                obuf_v[pl.ds((row + 2) * L + s * 16, 16)] = plsc.load_gather(c_v, [g3 + 2])
    pltpu.sync_copy(obuf_v, pts_out.at[wid])
    pltpu.sync_copy(tib_v, tib_out.at[wid])


def _dot0(t):
    return (t[0] + t[1]) + t[2]


def _dihedral_rows(pc):
    p0, p1, p2, p3 = pc[0:3], pc[3:6], pc[6:9], pc[9:12]
    b0 = [p0[i] - p1[i] for i in range(3)]
    b1 = [p2[i] - p1[i] for i in range(3)]
    b2 = [p3[i] - p2[i] for i in range(3)]
    ss = _dot0([b1[i] * b1[i] for i in range(3)])
    den = jnp.sqrt(ss) + jnp.float32(1e-8)
    b1n = [b1[i] / den for i in range(3)]
    s0 = _dot0([b0[i] * b1n[i] for i in range(3)])
    v = [b0[i] - s0 * b1n[i] for i in range(3)]
    s2 = _dot0([b2[i] * b1n[i] for i in range(3)])
    w = [b2[i] - s2 * b1n[i] for i in range(3)]
    x = _dot0([v[i] * w[i] for i in range(3)])
    cr = [b1n[(i + 1) % 3] * v[(i + 2) % 3] - b1n[(i + 2) % 3] * v[(i + 1) % 3]
          for i in range(3)]
    y = _dot0([cr[i] * w[i] for i in range(3)])
    return jnp.arctan2(y, x)


def _tc_math_body(pts_ref, tib_ref, wts_ref, gidx_ref):
    angs = []
    for t in range(2):
        pc = [pts_ref[0, t * 12 + r, :] for r in range(12)]
        angs.append(_dihedral_rows(pc))
    phi, psi = angs
    prm = [pts_ref[0, 24 + k, :] for k in range(4)]
    fi = (phi - prm[0]) / prm[2]
    fj = (psi - prm[1]) / prm[3]
    i0f = jnp.floor(fi)
    j0f = jnp.floor(fj)
    a = fi - i0f
    b = fj - j0f
    i0 = jnp.mod(i0f.astype(jnp.int32), BINS)
    j0 = jnp.mod(j0f.astype(jnp.int32), BINS)
    i1 = jnp.mod(i0 + 1, BINS)
    j1 = jnp.mod(j0 + 1, BINS)
    tib = tib_ref[0, 0, :]
    wts_ref[0, 0, :] = (1 - a) * (1 - b)
    wts_ref[0, 1, :] = a * (1 - b)
    wts_ref[0, 2, :] = (1 - a) * b
    wts_ref[0, 3, :] = a * b
    gidx_ref[0, 0, :] = tib + i0 * BINS + j0
    gidx_ref[0, 1, :] = tib + i1 * BINS + j0
    gidx_ref[0, 2, :] = tib + i0 * BINS + j1
    gidx_ref[0, 3, :] = tib + i1 * BINS + j1


def _sc_combine_body(gidx_hbm, wts_hbm, rama_hbm, out_hbm,
                     gi_v, wt_v, rama_v, out_v):
    cid = lax.axis_index("c")
    sid = lax.axis_index("s")
    wid = sid * 2 + cid
    pltpu.sync_copy(gidx_hbm.at[wid], gi_v)
    pltpu.sync_copy(wts_hbm.at[wid], wt_v)
    pltpu.sync_copy(rama_hbm, rama_v)
    acc = jnp.zeros((16,), jnp.float32)
    for s in range(NSTEP):
        vals = []
        for k in range(4):
            g = gi_v[pl.ds(k * L + s * 16, 16)]
            wk = wt_v[pl.ds(k * L + s * 16, 16)]
            vals.append(plsc.load_gather(rama_v, [g]) * wk)
        acc = acc + ((vals[0] + vals[1]) + (vals[2] + vals[3]))
    tot = jnp.sum(acc)
    out_v[...] = jnp.full((16,), tot, jnp.float32)
    pltpu.sync_copy(out_v, out_hbm.at[wid])


def kernel(coords, pose_stack_block_coord_offset, pose_stack_block_type,
           pose_stack_inter_residue_connections, bt_atom_downstream_of_conn,
           bt_rama_table, bt_upper_conn_ind, bt_is_pro, bt_rama_torsion_atoms,
           rama_tables, table_params):
    coords2 = coords.reshape(P, L * A * 3)
    offs2 = pose_stack_block_coord_offset.astype(jnp.int32)
    bt2 = pose_stack_block_type.astype(jnp.int32)
    irc2 = pose_stack_inter_residue_connections.astype(jnp.int32).reshape(P, L * 4)
    meta = jnp.concatenate([
        bt_upper_conn_ind.astype(jnp.int32),
        bt_is_pro.astype(jnp.int32),
        bt_rama_table.astype(jnp.int32).reshape(-1),
        bt_atom_downstream_of_conn.astype(jnp.int32).reshape(-1),
        bt_rama_torsion_atoms.astype(jnp.int32).reshape(-1),
    ])
    parflat = table_params.astype(jnp.float32).reshape(-1)
    ramaflat = rama_tables.astype(jnp.float32).reshape(-1)

    mesh = plsc.VectorSubcoreMesh(core_axis_name="c", subcore_axis_name="s",
                                  num_cores=2, num_subcores=16)

    gather_stage = pl.kernel(
        _sc_gather_body,
        out_type=(jax.ShapeDtypeStruct((P, 28 * L), jnp.float32),
                  jax.ShapeDtypeStruct((P, L), jnp.int32)),
        mesh=mesh,
        compiler_params=pltpu.CompilerParams(needs_layout_passes=False),
        scratch_types=[
            pltpu.VMEM((L * A * 3,), jnp.float32),
            pltpu.VMEM((L,), jnp.int32),
            pltpu.VMEM((L,), jnp.int32),
            pltpu.VMEM((L * 4,), jnp.int32),
            pltpu.VMEM((META_LEN,), jnp.int32),
            pltpu.VMEM((4 * N_TABLES,), jnp.float32),
            pltpu.VMEM((28 * L,), jnp.float32),
            pltpu.VMEM((L,), jnp.int32),
        ],
    )
    pts, tib = gather_stage(coords2, offs2, bt2, irc2, meta, parflat)

    wts, gidx = pl.pallas_call(
        _tc_math_body,
        grid=(P,),
        in_specs=[
            pl.BlockSpec((1, 28, L), lambda i: (i, 0, 0)),
            pl.BlockSpec((1, 1, L), lambda i: (i, 0, 0)),
        ],
        out_specs=[
            pl.BlockSpec((1, 4, L), lambda i: (i, 0, 0)),
            pl.BlockSpec((1, 4, L), lambda i: (i, 0, 0)),
        ],
        out_shape=[
            jax.ShapeDtypeStruct((P, 4, L), jnp.float32),
            jax.ShapeDtypeStruct((P, 4, L), jnp.int32),
        ],
    )(pts.reshape(P, 28, L), tib.reshape(P, 1, L))

    combine_stage = pl.kernel(
        _sc_combine_body,
        out_type=jax.ShapeDtypeStruct((P, 16), jnp.float32),
        mesh=mesh,
        compiler_params=pltpu.CompilerParams(needs_layout_passes=False),
        scratch_types=[
            pltpu.VMEM((4 * L,), jnp.int32),
            pltpu.VMEM((4 * L,), jnp.float32),
            pltpu.VMEM((N_TABLES * BINS * BINS,), jnp.float32),
            pltpu.VMEM((16,), jnp.float32),
        ],
    )
    out = combine_stage(gidx.reshape(P, 4 * L), wts.reshape(P, 4 * L), ramaflat)
    return out[:, 0]


# speedup vs baseline: 8.4555x; 1.2582x over previous
import jax
import jax.numpy as jnp
from jax import lax
from jax.experimental import pallas as pl
from jax.experimental.pallas import tpu as pltpu
from jax.experimental.pallas import tpu_sc as plsc

P, L, A = 32, 256, 28
T = 24
N_TABLES, BINS = 40, 36
NSTEP = L // 16
PL = P * L

OFF_UP = 0
OFF_PRO = OFF_UP + T
OFF_RTAB = OFF_PRO + T
OFF_DOWN = OFF_RTAB + 2 * T
OFF_TOR = OFF_DOWN + 2 * T * A
META_LEN = OFF_TOR + 8 * T


def _sc_gather_body(coords_hbm, offs_hbm, bt_hbm, irc_hbm, meta_hbm, par_hbm,
                    pts_out, tib_out,
                    c_v, offs_v, bt_v, irc_v, meta_v, par_v, obuf_v, tib_v):
    cid = lax.axis_index("c")
    sid = lax.axis_index("s")
    wid = sid * 2 + cid
    pltpu.sync_copy(coords_hbm.at[wid], c_v)
    pltpu.sync_copy(offs_hbm.at[wid], offs_v)
    pltpu.sync_copy(bt_hbm.at[wid], bt_v)
    pltpu.sync_copy(irc_hbm.at[wid], irc_v)
    pltpu.sync_copy(meta_hbm, meta_v)
    pltpu.sync_copy(par_hbm, par_v)

    iota = lax.iota(jnp.int32, 16)
    for s in range(NSTEP):
        sl = pl.ds(s * 16, 16)
        bt = bt_v[sl]
        off = offs_v[sl]
        up = plsc.load_gather(meta_v, [bt + OFF_UP])
        ib = (iota + s * 16) * 4 + up * 2
        nb = plsc.load_gather(irc_v, [ib])
        nc = plsc.load_gather(irc_v, [ib + 1])
        nbt = plsc.load_gather(bt_v, [nb])
        noff = plsc.load_gather(offs_v, [nb])
        down = plsc.load_gather(meta_v, [OFF_DOWN + (nbt * 2 + nc) * A])
        inter = noff + down
        ipro = plsc.load_gather(meta_v, [OFF_PRO + nbt])
        ti = plsc.load_gather(meta_v, [OFF_RTAB + bt * 2 + ipro])
        tib_v[sl] = ti * (BINS * BINS)
        p4 = ti * 4
        for k in range(4):
            obuf_v[24 + k, sl] = plsc.load_gather(par_v, [p4 + k])
        tor8 = bt * 8 + OFF_TOR
        for t in range(2):
            for j in range(4):
                ta = plsc.load_gather(meta_v, [tor8 + t * 4 + j])
                gi = jnp.where(ta >= 0, off + ta, inter)
                g3 = gi * 3
                row = (t * 4 + j) * 3
                obuf_v[row, sl] = plsc.load_gather(c_v, [g3])
                obuf_v[row + 1, sl] = plsc.load_gather(c_v, [g3 + 1])
                obuf_v[row + 2, sl] = plsc.load_gather(c_v, [g3 + 2])
    pltpu.sync_copy(obuf_v, pts_out.at[:, pl.ds(wid * L, L)])
    pltpu.sync_copy(tib_v, tib_out.at[pl.ds(wid * L, L)])


def _dot0(t):
    return (t[0] + t[1]) + t[2]


def _dihedral_rows(pc):
    p0, p1, p2, p3 = pc[0:3], pc[3:6], pc[6:9], pc[9:12]
    b0 = [p0[i] - p1[i] for i in range(3)]
    b1 = [p2[i] - p1[i] for i in range(3)]
    b2 = [p3[i] - p2[i] for i in range(3)]
    ss = _dot0([b1[i] * b1[i] for i in range(3)])
    den = jnp.sqrt(ss) + jnp.float32(1e-8)
    b1n = [b1[i] / den for i in range(3)]
    s0 = _dot0([b0[i] * b1n[i] for i in range(3)])
    v = [b0[i] - s0 * b1n[i] for i in range(3)]
    s2 = _dot0([b2[i] * b1n[i] for i in range(3)])
    w = [b2[i] - s2 * b1n[i] for i in range(3)]
    x = _dot0([v[i] * w[i] for i in range(3)])
    cr = [b1n[(i + 1) % 3] * v[(i + 2) % 3] - b1n[(i + 2) % 3] * v[(i + 1) % 3]
          for i in range(3)]
    y = _dot0([cr[i] * w[i] for i in range(3)])
    return jnp.arctan2(y, x)


def _tc_math_body(pts_ref, tib_ref, wts_ref, gidx_ref):
    angs = []
    for t in range(2):
        pc = [pts_ref[t * 12 + r, :] for r in range(12)]
        angs.append(_dihedral_rows(pc))
    phi, psi = angs
    prm = [pts_ref[24 + k, :] for k in range(4)]
    fi = (phi - prm[0]) / prm[2]
    fj = (psi - prm[1]) / prm[3]
    i0f = jnp.floor(fi)
    j0f = jnp.floor(fj)
    a = fi - i0f
    b = fj - j0f
    i0 = jnp.mod(i0f.astype(jnp.int32), BINS)
    j0 = jnp.mod(j0f.astype(jnp.int32), BINS)
    i1 = jnp.mod(i0 + 1, BINS)
    j1 = jnp.mod(j0 + 1, BINS)
    tib = tib_ref[0, :]
    wts_ref[0, :] = (1 - a) * (1 - b)
    wts_ref[1, :] = a * (1 - b)
    wts_ref[2, :] = (1 - a) * b
    wts_ref[3, :] = a * b
    gidx_ref[0, :] = tib + i0 * BINS + j0
    gidx_ref[1, :] = tib + i1 * BINS + j0
    gidx_ref[2, :] = tib + i0 * BINS + j1
    gidx_ref[3, :] = tib + i1 * BINS + j1


def _sc_combine_body(gidx_hbm, wts_hbm, rama_hbm, out_hbm,
                     gi_v, wt_v, vals_v, out_v, sem):
    cid = lax.axis_index("c")
    sid = lax.axis_index("s")
    wid = sid * 2 + cid
    for k in range(4):
        pltpu.sync_copy(gidx_hbm.at[k, pl.ds(wid * L, L)],
                        gi_v.at[pl.ds(k * L, L)])
        pltpu.sync_copy(wts_hbm.at[k, pl.ds(wid * L, L)],
                        wt_v.at[pl.ds(k * L, L)])
    pltpu.async_copy(rama_hbm.at[gi_v], vals_v, sem).wait()
    acc = jnp.zeros((16,), jnp.float32)
    for s in range(NSTEP):
        vals = []
        for k in range(4):
            vk = vals_v[pl.ds(k * L + s * 16, 16)]
            wk = wt_v[pl.ds(k * L + s * 16, 16)]
            vals.append(vk * wk)
        acc = acc + ((vals[0] + vals[1]) + (vals[2] + vals[3]))
    tot = jnp.sum(acc)
    out_v[...] = jnp.full((16,), tot, jnp.float32)
    pltpu.sync_copy(out_v, out_hbm.at[wid])


def kernel(coords, pose_stack_block_coord_offset, pose_stack_block_type,
           pose_stack_inter_residue_connections, bt_atom_downstream_of_conn,
           bt_rama_table, bt_upper_conn_ind, bt_is_pro, bt_rama_torsion_atoms,
           rama_tables, table_params):
    coords2 = coords.reshape(P, L * A * 3)
    offs2 = pose_stack_block_coord_offset.astype(jnp.int32)
    bt2 = pose_stack_block_type.astype(jnp.int32)
    irc2 = pose_stack_inter_residue_connections.astype(jnp.int32).reshape(P, L * 4)
    meta = jnp.concatenate([
        bt_upper_conn_ind.astype(jnp.int32),
        bt_is_pro.astype(jnp.int32),
        bt_rama_table.astype(jnp.int32).reshape(-1),
        bt_atom_downstream_of_conn.astype(jnp.int32).reshape(-1),
        bt_rama_torsion_atoms.astype(jnp.int32).reshape(-1),
    ])
    parflat = table_params.astype(jnp.float32).reshape(-1)
    ramaflat = rama_tables.astype(jnp.float32).reshape(-1)

    mesh = plsc.VectorSubcoreMesh(core_axis_name="c", subcore_axis_name="s",
                                  num_cores=2, num_subcores=16)

    gather_stage = pl.kernel(
        _sc_gather_body,
        out_type=(jax.ShapeDtypeStruct((28, PL), jnp.float32),
                  jax.ShapeDtypeStruct((PL,), jnp.int32)),
        mesh=mesh,
        compiler_params=pltpu.CompilerParams(needs_layout_passes=False),
        scratch_types=[
            pltpu.VMEM((L * A * 3,), jnp.float32),
            pltpu.VMEM((L,), jnp.int32),
            pltpu.VMEM((L,), jnp.int32),
            pltpu.VMEM((L * 4,), jnp.int32),
            pltpu.VMEM((META_LEN,), jnp.int32),
            pltpu.VMEM((4 * N_TABLES,), jnp.float32),
            pltpu.VMEM((28, L), jnp.float32),
            pltpu.VMEM((L,), jnp.int32),
        ],
    )
    pts, tib = gather_stage(coords2, offs2, bt2, irc2, meta, parflat)

    wts, gidx = pl.pallas_call(
        _tc_math_body,
        out_shape=[
            jax.ShapeDtypeStruct((4, PL), jnp.float32),
            jax.ShapeDtypeStruct((4, PL), jnp.int32),
        ],
    )(pts, tib.reshape(1, PL))

    combine_stage = pl.kernel(
        _sc_combine_body,
        out_type=jax.ShapeDtypeStruct((P, 16), jnp.float32),
        mesh=mesh,
        compiler_params=pltpu.CompilerParams(needs_layout_passes=False),
        scratch_types=[
            pltpu.VMEM((4 * L,), jnp.int32),
            pltpu.VMEM((4 * L,), jnp.float32),
            pltpu.VMEM((4 * L,), jnp.float32),
            pltpu.VMEM((16,), jnp.float32),
            pltpu.SemaphoreType.DMA,
        ],
    )
    out = combine_stage(gidx, wts, ramaflat)
    return out[:, 0]
